# Initial kernel scaffold; baseline (speedup 1.0000x reference)
import jax
import jax.numpy as jnp
from jax.experimental import pallas as pl


def kernel(x, weight):
    raise NotImplementedError("write your pallas kernel here")


# speedup vs baseline: 4.5681x; 4.5681x over previous
import functools

import jax
import jax.numpy as jnp
from jax import lax
from jax.experimental import pallas as pl
from jax.experimental.pallas import tpu as pltpu
from jax.experimental.pallas import tpu_sc as plsc

_D = 64
_NW = 32
_IDX_ROW = 128
_ROWS_PER_CHUNK = 640
_GATHERS = _ROWS_PER_CHUNK // _IDX_ROW


@functools.partial(jax.jit, static_argnums=(2,))
def _sc_embedding_gather(idx3d, weight, b_total):
    b_per_w = b_total // _NW
    idx_rows_per_w = b_per_w // _IDX_ROW
    n_chunks = b_per_w // _ROWS_PER_CHUNK
    mesh = plsc.VectorSubcoreMesh(core_axis_name="c", subcore_axis_name="s")

    @functools.partial(
        pl.kernel,
        out_type=jax.ShapeDtypeStruct((b_total, _D), jnp.float32),
        mesh=mesh,
        scratch_types=[
            pltpu.VMEM((idx_rows_per_w, _IDX_ROW), jnp.int32),
            pltpu.VMEM((_ROWS_PER_CHUNK, _D), jnp.float32),
            pltpu.SemaphoreType.DMA,
        ],
        compiler_params=pltpu.CompilerParams(use_tc_tiling_on_sc=False),
    )
    def k(table_hbm, idx_hbm, out_hbm, idx_v, rows_v, gsem):
        wid = lax.axis_index("s") * 2 + lax.axis_index("c")
        base = wid * b_per_w
        pltpu.sync_copy(idx_hbm.at[wid], idx_v)

        def chunk_body(c):
            descs = []
            for j in range(_GATHERS):
                descs.append(
                    pltpu.async_copy(
                        table_hbm.at[idx_v.at[c * _GATHERS + j]],
                        rows_v.at[pl.ds(j * _IDX_ROW, _IDX_ROW)],
                        gsem,
                    )
                )
            for d in descs:
                d.wait()
            pltpu.sync_copy(
                rows_v,
                out_hbm.at[pl.ds(base + c * _ROWS_PER_CHUNK, _ROWS_PER_CHUNK)],
            )

        lax.fori_loop(0, n_chunks, lambda c, _: (chunk_body(c), 0)[1], 0)

    return k(weight, idx3d)


def kernel(x, weight):
    b, s = x.shape
    b_total = b * s
    idx3d = x.reshape(_NW, (b_total // _NW) // _IDX_ROW, _IDX_ROW).astype(
        jnp.int32
    )
    out = _sc_embedding_gather(idx3d, weight, b_total)
    return out.reshape(b, s, _D)


# speedup vs baseline: 4.6875x; 1.0261x over previous
import functools

import jax
import jax.numpy as jnp
from jax import lax
from jax.experimental import pallas as pl
from jax.experimental.pallas import tpu as pltpu
from jax.experimental.pallas import tpu_sc as plsc

_D = 64
_NW = 32
_IDX_ROW = 128
_ROWS_PER_CHUNK = 640
_IDXR_PER_CHUNK = _ROWS_PER_CHUNK // _IDX_ROW
_NSLOT = 3


@functools.partial(jax.jit, static_argnums=(2,))
def _sc_embedding_gather(idx2d, weight, b_total):
    b_per_w = b_total // _NW
    n_chunks = b_per_w // _ROWS_PER_CHUNK
    mesh = plsc.VectorSubcoreMesh(core_axis_name="c", subcore_axis_name="s")

    @functools.partial(
        pl.kernel,
        out_type=jax.ShapeDtypeStruct((b_total, _D), jnp.float32),
        mesh=mesh,
        scratch_types=[
            pltpu.VMEM((b_per_w,), jnp.int32),
            pltpu.VMEM((_NSLOT, _ROWS_PER_CHUNK, _D), jnp.float32),
            pltpu.SemaphoreType.DMA((_NSLOT,)),
            pltpu.SemaphoreType.DMA((_NSLOT,)),
        ],
        compiler_params=pltpu.CompilerParams(use_tc_tiling_on_sc=False),
    )
    def k(table_hbm, idx_hbm, out_hbm, idx_v, rows_v, gsem, ssem):
        wid = lax.axis_index("s") * 2 + lax.axis_index("c")
        base = wid * b_per_w
        pltpu.sync_copy(idx_hbm.at[wid], idx_v)

        def fire_gather(c, slot):
            return pltpu.async_copy(
                table_hbm.at[idx_v.at[pl.ds(c * _ROWS_PER_CHUNK, _ROWS_PER_CHUNK)]],
                rows_v.at[slot],
                gsem.at[slot],
            )

        def fire_scatter(c, slot):
            return pltpu.async_copy(
                rows_v.at[slot],
                out_hbm.at[pl.ds(base + c * _ROWS_PER_CHUNK, _ROWS_PER_CHUNK)],
                ssem.at[slot],
            )

        gd = {}
        sd = {}
        for c in range(_NSLOT):
            gd[c] = fire_gather(c, c % _NSLOT)
        for c in range(n_chunks):
            slot = c % _NSLOT
            gd[c].wait()
            sd[c] = fire_scatter(c, slot)
            nxt = c + _NSLOT
            if nxt < n_chunks:
                sd[c].wait()
                gd[nxt] = fire_gather(nxt, slot)
        for c in range(n_chunks - _NSLOT, n_chunks):
            sd[c].wait()

    return k(weight, idx2d)


def kernel(x, weight):
    b, s = x.shape
    b_total = b * s
    idx2d = x.reshape(_NW, b_total // _NW).astype(jnp.int32)
    out = _sc_embedding_gather(idx2d, weight, b_total)
    return out.reshape(b, s, _D)
